# Initial kernel scaffold; baseline (speedup 1.0000x reference)
import jax
import jax.numpy as jnp
from jax.experimental import pallas as pl


def kernel(x, position_weight, value_weight):
    raise NotImplementedError("write your pallas kernel here")


# speedup vs baseline: 13.2388x; 13.2388x over previous
import jax
import jax.numpy as jnp
from jax.experimental import pallas as pl

_OUT_FEATURES = 4096
_SIZE = 26
_LEVELS = 100
_LOW = 0.0
_HIGH = 1.0


def _encode_kernel(x_ref, pos_ref, val_ref, out_ref):
    x = x_ref[...]
    idx = jnp.clip(
        jnp.round((x - _LOW) / (_HIGH - _LOW) * (_LEVELS - 1)), 0, _LEVELS - 1
    ).astype(jnp.int32)

    vs = (1 - 2 * val_ref[...].astype(jnp.int32)).astype(jnp.bfloat16)
    ps = (1 - 2 * pos_ref[...].astype(jnp.int32)).astype(jnp.bfloat16)

    b = x.shape[0]
    lanes = jax.lax.broadcasted_iota(jnp.int32, (b, _LEVELS), 1)
    t = jnp.zeros((b, _OUT_FEATURES), jnp.float32)
    for s in range(_SIZE):
        m_s = (idx[:, s : s + 1] == lanes).astype(jnp.bfloat16)
        w_s = ps[s : s + 1, :] * vs
        t = t + jnp.dot(m_s, w_s, preferred_element_type=jnp.float32)
    out_ref[...] = (t <= 0.0).astype(jnp.uint8)


def kernel(x, position_weight, value_weight):
    batch = x.shape[0]
    return pl.pallas_call(
        _encode_kernel,
        out_shape=jax.ShapeDtypeStruct((batch, _OUT_FEATURES), jnp.uint8),
    )(x, position_weight, value_weight)
